# Initial kernel scaffold; baseline (speedup 1.0000x reference)
import jax
import jax.numpy as jnp
from jax.experimental import pallas as pl


def kernel(x_q, x_edge, edge_ids, w_q):
    raise NotImplementedError("write your pallas kernel here")


# speedup vs baseline: 1.9358x; 1.9358x over previous
import functools

import jax
import jax.numpy as jnp
from jax import lax
from jax.experimental import pallas as pl
from jax.experimental.pallas import tpu as pltpu
from jax.experimental.pallas import tpu_sc as plsc

NP = 640000
NE = 320000
D = 128
ALPHA_C = 0.5

WE = 512
NWIN = NE // WE
NC, NS = 2, 16
NWORK = NC * NS
WPW = (NWIN + NWORK - 1) // NWORK
C = 128
ROWS = WE + 8
ZR = ROWS // 4


def _sc_segment_sum(x_q, ids, p2, zrows):
    mesh = plsc.VectorSubcoreMesh(core_axis_name="c", subcore_axis_name="s")

    @functools.partial(
        pl.kernel,
        out_type=jax.ShapeDtypeStruct((NE, D), jnp.float32),
        mesh=mesh,
        scratch_types=[
            pltpu.VMEM((C, D), jnp.float32),
            pltpu.VMEM((C,), jnp.int32),
            pltpu.VMEM((C,), jnp.int32),
            pltpu.VMEM((2, 16), jnp.int32),
            pltpu.VMEM((ZR, D), jnp.float32),
            pltpu.VMEM_SHARED((NS * ROWS, D), jnp.float32),
        ],
    )
    def k(xq_hbm, ids_hbm, p2_hbm, z_hbm, agg_hbm,
          x_buf, id_buf, idx_buf, pvec, zbuf, acc):
        cid = lax.axis_index("c")
        sid = lax.axis_index("s")
        w = cid * NS + sid
        base_row = sid * ROWS
        dump = base_row + WE
        pltpu.sync_copy(z_hbm, zbuf)

        def window_body(kk, carry):
            win = w + NWORK * kk

            @pl.when(win < NWIN)
            def _():
                e0 = win * WE
                pltpu.sync_copy(p2_hbm.at[win], pvec)
                p_start = pvec[0, :][0]
                p_end = pvec[1, :][0]
                a0 = (p_start // C) * C
                nchunks = jnp.where(p_end > p_start,
                                    (p_end - a0 + C - 1) // C, 0)
                for r in range(ROWS // ZR):
                    pltpu.sync_copy(zbuf, acc.at[pl.ds(base_row + r * ZR, ZR)])

                def chunk_body(j, carry2):
                    p = a0 + j * C
                    pc = pl.multiple_of(jnp.minimum(p, NP - C), C)
                    pltpu.sync_copy(ids_hbm.at[pl.ds(pc, C)], id_buf)
                    pltpu.sync_copy(xq_hbm.at[pl.ds(pc, C)], x_buf)
                    lo = jnp.maximum(p, p_start)
                    for g in range(C // 16):
                        idv = id_buf[pl.ds(g * 16, 16)]
                        gi = pc + g * 16 + lax.iota(jnp.int32, 16)
                        valid = (gi >= lo) & (gi < p_end)
                        loc = jnp.where(valid, idv - e0 + base_row, dump)
                        idx_buf[pl.ds(g * 16, 16)] = loc
                    pltpu.sync_copy(x_buf, acc.at[idx_buf], add=True)
                    return carry2

                lax.fori_loop(0, nchunks, chunk_body, 0)
                pltpu.sync_copy(acc.at[pl.ds(base_row, WE)],
                                agg_hbm.at[pl.ds(e0, WE)])

            return carry

        lax.fori_loop(0, WPW, window_body, 0)

    return k(x_q, ids, p2, zrows)


def _tc_combine(agg, x_edge, w_q):
    R = 512

    def body(agg_ref, xe_ref, wq_ref, out_ref):
        out_ref[...] = ALPHA_C * xe_ref[...] + (1.0 - ALPHA_C) * jnp.dot(
            agg_ref[...], wq_ref[...], preferred_element_type=jnp.float32)

    return pl.pallas_call(
        body,
        grid=(NE // R,),
        in_specs=[
            pl.BlockSpec((R, D), lambda i: (i, 0)),
            pl.BlockSpec((R, D), lambda i: (i, 0)),
            pl.BlockSpec((D, D), lambda i: (0, 0)),
        ],
        out_specs=pl.BlockSpec((R, D), lambda i: (i, 0)),
        out_shape=jax.ShapeDtypeStruct((NE, D), jnp.float32),
    )(agg, x_edge, w_q)


def kernel(x_q, x_edge, edge_ids, w_q):
    ids = edge_ids.astype(jnp.int32)
    bounds = jnp.arange(NWIN + 1, dtype=jnp.int32) * WE
    p = jnp.searchsorted(ids, bounds, side="left").astype(jnp.int32)
    p2 = jnp.zeros((NWIN, 2, 16), jnp.int32)
    p2 = p2.at[:, 0, 0].set(p[:-1]).at[:, 1, 0].set(p[1:])
    z = jnp.zeros((ZR, D), jnp.float32)
    agg = _sc_segment_sum(x_q, ids, p2, z)
    return _tc_combine(agg, x_edge, w_q)


# speedup vs baseline: 2.3915x; 1.2354x over previous
import functools

import jax
import jax.numpy as jnp
from jax import lax
from jax.experimental import pallas as pl
from jax.experimental.pallas import tpu as pltpu
from jax.experimental.pallas import tpu_sc as plsc

NP = 640000
NE = 320000
D = 128
ALPHA_C = 0.5

WE = 512
NWIN = NE // WE
NC, NS = 2, 16
NWORK = NC * NS
WPW = (NWIN + NWORK - 1) // NWORK
C = 128
ROWS = WE + 8
ZR = ROWS // 4


def _sc_segment_sum(x_q, ids, p2, zrows):
    mesh = plsc.VectorSubcoreMesh(core_axis_name="c", subcore_axis_name="s")

    @functools.partial(
        pl.kernel,
        out_type=jax.ShapeDtypeStruct((NE, D), jnp.float32),
        mesh=mesh,
        scratch_types=[
            pltpu.VMEM((C, D), jnp.float32),
            pltpu.VMEM((C,), jnp.int32),
            pltpu.VMEM((C,), jnp.int32),
            pltpu.VMEM((2, 16), jnp.int32),
            pltpu.VMEM((ZR, D), jnp.float32),
            pltpu.VMEM_SHARED((NS * ROWS, D), jnp.float32),
        ],
    )
    def k(xq_hbm, ids_hbm, p2_hbm, z_hbm, agg_hbm,
          x_buf, id_buf, idx_buf, pvec, zbuf, acc):
        cid = lax.axis_index("c")
        sid = lax.axis_index("s")
        w = cid * NS + sid
        base_row = sid * ROWS
        dump = base_row + WE
        pltpu.sync_copy(z_hbm, zbuf)

        def window_body(kk, carry):
            win = w + NWORK * kk

            @pl.when(win < NWIN)
            def _():
                e0 = win * WE
                pltpu.sync_copy(p2_hbm.at[win], pvec)
                p_start = pvec[0, :][0]
                p_end = pvec[1, :][0]
                a0 = (p_start // C) * C
                nchunks = jnp.where(p_end > p_start,
                                    (p_end - a0 + C - 1) // C, 0)
                for r in range(ROWS // ZR):
                    pltpu.sync_copy(zbuf, acc.at[pl.ds(base_row + r * ZR, ZR)])

                def chunk_body(j, carry2):
                    p = a0 + j * C
                    pc = pl.multiple_of(jnp.minimum(p, NP - C), C)
                    pltpu.sync_copy(ids_hbm.at[pl.ds(pc, C)], id_buf)
                    pltpu.sync_copy(xq_hbm.at[pl.ds(pc, C)], x_buf)
                    lo = jnp.maximum(p, p_start)
                    for g in range(C // 16):
                        idv = id_buf[pl.ds(g * 16, 16)]
                        gi = pc + g * 16 + lax.iota(jnp.int32, 16)
                        valid = (gi >= lo) & (gi < p_end)
                        loc = jnp.where(valid, idv - e0 + base_row, dump)
                        idx_buf[pl.ds(g * 16, 16)] = loc
                    pltpu.sync_copy(x_buf, acc.at[idx_buf], add=True)
                    return carry2

                lax.fori_loop(0, nchunks, chunk_body, 0)
                pltpu.sync_copy(acc.at[pl.ds(base_row, WE)],
                                agg_hbm.at[pl.ds(e0, WE)])

            return carry

        lax.fori_loop(0, WPW, window_body, 0)

    return k(x_q, ids, p2, zrows)


def _tc_combine(agg, x_edge, w_q):
    R = 512

    def body(agg_ref, xe_ref, wq_ref, out_ref):
        out_ref[...] = ALPHA_C * xe_ref[...] + (1.0 - ALPHA_C) * jnp.dot(
            agg_ref[...], wq_ref[...], preferred_element_type=jnp.float32)

    return pl.pallas_call(
        body,
        grid=(NE // R,),
        in_specs=[
            pl.BlockSpec((R, D), lambda i: (i, 0)),
            pl.BlockSpec((R, D), lambda i: (i, 0)),
            pl.BlockSpec((D, D), lambda i: (0, 0)),
        ],
        out_specs=pl.BlockSpec((R, D), lambda i: (i, 0)),
        out_shape=jax.ShapeDtypeStruct((NE, D), jnp.float32),
    )(agg, x_edge, w_q)


def kernel(x_q, x_edge, edge_ids, w_q):
    ids = edge_ids.astype(jnp.int32)
    bounds = jnp.arange(NWIN + 1, dtype=jnp.int32) * WE
    stride = 1024
    s = ids[::stride]
    k = jnp.sum(s[None, :] < bounds[:, None], axis=1).astype(jnp.int32)
    w0 = jnp.clip((k - 1) * stride, 0, NP - stride)
    wnd = ids[w0[:, None] + jnp.arange(stride, dtype=jnp.int32)[None, :]]
    p = (w0 + jnp.sum(wnd < bounds[:, None], axis=1)).astype(jnp.int32)
    p2 = jnp.stack([
        jnp.broadcast_to(p[:-1, None], (NWIN, 16)),
        jnp.broadcast_to(p[1:, None], (NWIN, 16)),
    ], axis=1)
    z = jnp.zeros((ZR, D), jnp.float32)
    agg = _sc_segment_sum(x_q, ids, p2, z)
    return _tc_combine(agg, x_edge, w_q)


# speedup vs baseline: 2.9971x; 1.2532x over previous
import functools

import jax
import jax.numpy as jnp
from jax import lax
from jax.experimental import pallas as pl
from jax.experimental.pallas import tpu as pltpu
from jax.experimental.pallas import tpu_sc as plsc

NP = 640000
NE = 320000
D = 128
ALPHA_C = 0.5

WE = 512
NWIN = NE // WE
NC, NS = 2, 16
NWORK = NC * NS
WPW = (NWIN + NWORK - 1) // NWORK
C = 128
ROWS = WE + 8
ZR = ROWS // 4


def _sc_segment_sum(x_q, ids, p2, zrows):
    mesh = plsc.VectorSubcoreMesh(core_axis_name="c", subcore_axis_name="s")

    @functools.partial(
        pl.kernel,
        out_type=jax.ShapeDtypeStruct((NE, D), jnp.float32),
        mesh=mesh,
        scratch_types=[
            pltpu.VMEM((C, D), jnp.float32),
            pltpu.VMEM((C, D), jnp.float32),
            pltpu.VMEM((C,), jnp.int32),
            pltpu.VMEM((C,), jnp.int32),
            pltpu.VMEM((C,), jnp.int32),
            pltpu.VMEM((C,), jnp.int32),
            pltpu.VMEM((2, 16), jnp.int32),
            pltpu.VMEM((ZR, D), jnp.float32),
            pltpu.VMEM_SHARED((NS * ROWS, D), jnp.float32),
            pltpu.SemaphoreType.DMA,
            pltpu.SemaphoreType.DMA,
            pltpu.SemaphoreType.DMA,
            pltpu.SemaphoreType.DMA,
        ],
    )
    def k(xq_hbm, ids_hbm, p2_hbm, z_hbm, agg_hbm,
          x0, x1, id0, id1, ix0, ix1, pvec, zbuf, acc,
          s_id0, s_id1, s_x0, s_x1):
        cid = lax.axis_index("c")
        sid = lax.axis_index("s")
        w = cid * NS + sid
        base_row = sid * ROWS
        dump = base_row + WE
        pltpu.sync_copy(z_hbm, zbuf)
        bufs = ((x0, id0, ix0, s_x0, s_id0), (x1, id1, ix1, s_x1, s_id1))

        def window_body(kk, carry):
            win = w + NWORK * kk

            @pl.when(win < NWIN)
            def _():
                e0 = win * WE
                pltpu.sync_copy(p2_hbm.at[win], pvec)
                p_start = pvec[0, :][0]
                p_end = pvec[1, :][0]
                a0 = (p_start // C) * C
                nchunks = jnp.where(p_end > p_start,
                                    (p_end - a0 + C - 1) // C, 0)
                for r in range(ROWS // ZR):
                    pltpu.sync_copy(zbuf, acc.at[pl.ds(base_row + r * ZR, ZR)])

                def chunk_pc(j):
                    p = a0 + j * C
                    return p, pl.multiple_of(jnp.minimum(p, NP - C), C)

                def start_load(j, b):
                    xb, idb, _, sx, sid_sem = bufs[b]
                    _, pc = chunk_pc(j)
                    pltpu.async_copy(ids_hbm.at[pl.ds(pc, C)], idb, sid_sem)
                    pltpu.async_copy(xq_hbm.at[pl.ds(pc, C)], xb, sx)

                def process(j, b):
                    xb, idb, ixb, sx, sid_sem = bufs[b]
                    p, pc = chunk_pc(j)
                    pltpu.make_async_copy(ids_hbm.at[pl.ds(0, C)], idb,
                                          sid_sem).wait()
                    lo = jnp.maximum(p, p_start)
                    for g in range(C // 16):
                        idv = idb[pl.ds(g * 16, 16)]
                        gi = pc + g * 16 + lax.iota(jnp.int32, 16)
                        valid = (gi >= lo) & (gi < p_end)
                        loc = jnp.where(valid, idv - e0 + base_row, dump)
                        ixb[pl.ds(g * 16, 16)] = loc
                    pltpu.make_async_copy(xq_hbm.at[pl.ds(0, C)], xb, sx).wait()
                    pltpu.sync_copy(xb, acc.at[ixb], add=True)

                @pl.when(nchunks > 0)
                def _():
                    start_load(0, 0)

                    def pair_body(t, carry2):
                        c0 = 2 * t
                        c1 = c0 + 1
                        c2 = c0 + 2

                        @pl.when(c1 < nchunks)
                        def _():
                            start_load(c1, 1)

                        process(c0, 0)

                        @pl.when(c2 < nchunks)
                        def _():
                            start_load(c2, 0)

                        @pl.when(c1 < nchunks)
                        def _():
                            process(c1, 1)

                        return carry2

                    lax.fori_loop(0, (nchunks + 1) // 2, pair_body, 0)

                pltpu.sync_copy(acc.at[pl.ds(base_row, WE)],
                                agg_hbm.at[pl.ds(e0, WE)])

            return carry

        lax.fori_loop(0, WPW, window_body, 0)

    return k(x_q, ids, p2, zrows)


def _tc_combine(agg, x_edge, w_q):
    R = 512

    def body(agg_ref, xe_ref, wq_ref, out_ref):
        out_ref[...] = ALPHA_C * xe_ref[...] + (1.0 - ALPHA_C) * jnp.dot(
            agg_ref[...], wq_ref[...], preferred_element_type=jnp.float32)

    return pl.pallas_call(
        body,
        grid=(NE // R,),
        in_specs=[
            pl.BlockSpec((R, D), lambda i: (i, 0)),
            pl.BlockSpec((R, D), lambda i: (i, 0)),
            pl.BlockSpec((D, D), lambda i: (0, 0)),
        ],
        out_specs=pl.BlockSpec((R, D), lambda i: (i, 0)),
        out_shape=jax.ShapeDtypeStruct((NE, D), jnp.float32),
    )(agg, x_edge, w_q)


def kernel(x_q, x_edge, edge_ids, w_q):
    ids = edge_ids.astype(jnp.int32)
    bounds = jnp.arange(NWIN + 1, dtype=jnp.int32) * WE
    stride = 1024
    s = ids[::stride]
    k = jnp.sum(s[None, :] < bounds[:, None], axis=1).astype(jnp.int32)
    w0 = jnp.clip((k - 1) * stride, 0, NP - stride)
    wnd = ids[w0[:, None] + jnp.arange(stride, dtype=jnp.int32)[None, :]]
    p = (w0 + jnp.sum(wnd < bounds[:, None], axis=1)).astype(jnp.int32)
    p2 = jnp.stack([
        jnp.broadcast_to(p[:-1, None], (NWIN, 16)),
        jnp.broadcast_to(p[1:, None], (NWIN, 16)),
    ], axis=1)
    z = jnp.zeros((ZR, D), jnp.float32)
    agg = _sc_segment_sum(x_q, ids, p2, z)
    return _tc_combine(agg, x_edge, w_q)


# speedup vs baseline: 3.4904x; 1.1646x over previous
import functools

import jax
import jax.numpy as jnp
from jax import lax
from jax.experimental import pallas as pl
from jax.experimental.pallas import tpu as pltpu
from jax.experimental.pallas import tpu_sc as plsc

NP = 640000
NE = 320000
D = 128
ALPHA_C = 0.5

WE = 512
NWIN = NE // WE
NC, NS = 2, 16
NWORK = NC * NS
WPW = (NWIN + NWORK - 1) // NWORK
C = 128
ROWS = WE + 8
ZR = ROWS // 4


def _sc_segment_sum(x_q, ids, p2, zrows, wlo, span):
    mesh = plsc.VectorSubcoreMesh(core_axis_name="c", subcore_axis_name="s")
    wpw = (span + NWORK - 1) // NWORK

    @functools.partial(
        pl.kernel,
        out_type=jax.ShapeDtypeStruct((span * WE, D), jnp.float32),
        mesh=mesh,
        scratch_types=[
            pltpu.VMEM((C, D), jnp.float32),
            pltpu.VMEM((C, D), jnp.float32),
            pltpu.VMEM((C,), jnp.int32),
            pltpu.VMEM((C,), jnp.int32),
            pltpu.VMEM((C,), jnp.int32),
            pltpu.VMEM((C,), jnp.int32),
            pltpu.VMEM((2, 16), jnp.int32),
            pltpu.VMEM((ZR, D), jnp.float32),
            pltpu.VMEM_SHARED((NS * ROWS, D), jnp.float32),
            pltpu.SemaphoreType.DMA,
            pltpu.SemaphoreType.DMA,
            pltpu.SemaphoreType.DMA,
            pltpu.SemaphoreType.DMA,
        ],
    )
    def k(xq_hbm, ids_hbm, p2_hbm, z_hbm, agg_hbm,
          x0, x1, id0, id1, ix0, ix1, pvec, zbuf, acc,
          s_id0, s_id1, s_x0, s_x1):
        cid = lax.axis_index("c")
        sid = lax.axis_index("s")
        w = cid * NS + sid
        base_row = sid * ROWS
        dump = base_row + WE
        pltpu.sync_copy(z_hbm, zbuf)
        bufs = ((x0, id0, ix0, s_x0, s_id0), (x1, id1, ix1, s_x1, s_id1))

        def window_body(kk, carry):
            win = wlo + w + NWORK * kk

            @pl.when(win < wlo + span)
            def _():
                e0 = win * WE
                pltpu.sync_copy(p2_hbm.at[win], pvec)
                p_start = pvec[0, :][0]
                p_end = pvec[1, :][0]
                a0 = (p_start // C) * C
                nchunks = jnp.where(p_end > p_start,
                                    (p_end - a0 + C - 1) // C, 0)
                for r in range(ROWS // ZR):
                    pltpu.sync_copy(zbuf, acc.at[pl.ds(base_row + r * ZR, ZR)])

                def chunk_pc(j):
                    p = a0 + j * C
                    return p, pl.multiple_of(jnp.minimum(p, NP - C), C)

                def start_load(j, b):
                    xb, idb, _, sx, sid_sem = bufs[b]
                    _, pc = chunk_pc(j)
                    pltpu.async_copy(ids_hbm.at[pl.ds(pc, C)], idb, sid_sem)
                    pltpu.async_copy(xq_hbm.at[pl.ds(pc, C)], xb, sx)

                def process(j, b):
                    xb, idb, ixb, sx, sid_sem = bufs[b]
                    p, pc = chunk_pc(j)
                    pltpu.make_async_copy(ids_hbm.at[pl.ds(0, C)], idb,
                                          sid_sem).wait()
                    lo = jnp.maximum(p, p_start)
                    for g in range(C // 16):
                        idv = idb[pl.ds(g * 16, 16)]
                        gi = pc + g * 16 + lax.iota(jnp.int32, 16)
                        valid = (gi >= lo) & (gi < p_end)
                        loc = jnp.where(valid, idv - e0 + base_row, dump)
                        ixb[pl.ds(g * 16, 16)] = loc
                    pltpu.make_async_copy(xq_hbm.at[pl.ds(0, C)], xb, sx).wait()
                    pltpu.sync_copy(xb, acc.at[ixb], add=True)

                @pl.when(nchunks > 0)
                def _():
                    start_load(0, 0)

                    def pair_body(t, carry2):
                        c0 = 2 * t
                        c1 = c0 + 1
                        c2 = c0 + 2

                        @pl.when(c1 < nchunks)
                        def _():
                            start_load(c1, 1)

                        process(c0, 0)

                        @pl.when(c2 < nchunks)
                        def _():
                            start_load(c2, 0)

                        @pl.when(c1 < nchunks)
                        def _():
                            process(c1, 1)

                        return carry2

                    lax.fori_loop(0, (nchunks + 1) // 2, pair_body, 0)

                pltpu.sync_copy(acc.at[pl.ds(base_row, WE)],
                                agg_hbm.at[pl.ds(e0 - wlo * WE, WE)])

            return carry

        lax.fori_loop(0, wpw, window_body, 0)

    return k(x_q, ids, p2, zrows)


def _tc_combine_slice(agg, x_edge, w_q, row0, nrows, prev):
    R = 512
    off = row0 // R

    def body(agg_ref, xe_ref, wq_ref, prev_ref, out_ref):
        del prev_ref
        out_ref[...] = ALPHA_C * xe_ref[...] + (1.0 - ALPHA_C) * jnp.dot(
            agg_ref[...], wq_ref[...], preferred_element_type=jnp.float32)

    return pl.pallas_call(
        body,
        grid=(nrows // R,),
        in_specs=[
            pl.BlockSpec((R, D), lambda i: (i, 0)),
            pl.BlockSpec((R, D), lambda i, _off=off: (i + _off, 0)),
            pl.BlockSpec((D, D), lambda i: (0, 0)),
            pl.BlockSpec(memory_space=pl.ANY),
        ],
        out_specs=pl.BlockSpec((R, D), lambda i, _off=off: (i + _off, 0)),
        out_shape=jax.ShapeDtypeStruct((NE, D), jnp.float32),
        input_output_aliases={3: 0},
    )(agg, x_edge, w_q, prev)


def kernel(x_q, x_edge, edge_ids, w_q):
    ids = edge_ids.astype(jnp.int32)
    bounds = jnp.arange(NWIN + 1, dtype=jnp.int32) * WE
    stride = 1024
    s = ids[::stride]
    k = jnp.sum(s[None, :] < bounds[:, None], axis=1).astype(jnp.int32)
    w0 = jnp.clip((k - 1) * stride, 0, NP - stride)
    wnd = ids[w0[:, None] + jnp.arange(stride, dtype=jnp.int32)[None, :]]
    p = (w0 + jnp.sum(wnd < bounds[:, None], axis=1)).astype(jnp.int32)
    p2 = jnp.stack([
        jnp.broadcast_to(p[:-1, None], (NWIN, 16)),
        jnp.broadcast_to(p[1:, None], (NWIN, 16)),
    ], axis=1)
    z = jnp.zeros((ZR, D), jnp.float32)
    splits = [0, 156, 312, 468, NWIN]
    aggs = [
        _sc_segment_sum(x_q, ids, p2, z, splits[j], splits[j + 1] - splits[j])
        for j in range(len(splits) - 1)
    ]
    out = jnp.empty((NE, D), jnp.float32)
    for j, agg in enumerate(aggs):
        out = _tc_combine_slice(agg, x_edge, w_q, splits[j] * WE,
                                (splits[j + 1] - splits[j]) * WE, out)
    return out


# speedup vs baseline: 4.6020x; 1.3185x over previous
import functools

import jax
import jax.numpy as jnp
from jax import lax
from jax.experimental import pallas as pl
from jax.experimental.pallas import tpu as pltpu
from jax.experimental.pallas import tpu_sc as plsc

NP = 640000
NE = 320000
D = 128
ALPHA_C = 0.5

WE = 512
NWIN = NE // WE
NC, NS = 2, 16
NWORK = NC * NS
WPW = (NWIN + NWORK - 1) // NWORK
C = 128
ROWS = WE + 8
ZR = ROWS // 4


def _sc_segment_sum(x_q, ids, p2, zrows, wlo, span):
    mesh = plsc.VectorSubcoreMesh(core_axis_name="c", subcore_axis_name="s")
    wpw = (span + NWORK - 1) // NWORK

    @functools.partial(
        pl.kernel,
        out_type=jax.ShapeDtypeStruct((span * WE, D), jnp.float32),
        mesh=mesh,
        scratch_types=[
            pltpu.VMEM((C, D), jnp.float32),
            pltpu.VMEM((C, D), jnp.float32),
            pltpu.VMEM((C,), jnp.int32),
            pltpu.VMEM((C,), jnp.int32),
            pltpu.VMEM((C,), jnp.int32),
            pltpu.VMEM((C,), jnp.int32),
            pltpu.VMEM((2, 16), jnp.int32),
            pltpu.VMEM((ZR, D), jnp.float32),
            pltpu.VMEM_SHARED((NS * ROWS, D), jnp.float32),
            pltpu.SemaphoreType.DMA,
            pltpu.SemaphoreType.DMA,
            pltpu.SemaphoreType.DMA,
            pltpu.SemaphoreType.DMA,
        ],
    )
    def k(xq_hbm, ids_hbm, p2_hbm, z_hbm, agg_hbm,
          x0, x1, id0, id1, ix0, ix1, pvec, zbuf, acc,
          s_id0, s_id1, s_x0, s_x1):
        cid = lax.axis_index("c")
        sid = lax.axis_index("s")
        w = cid * NS + sid
        base_row = sid * ROWS
        dump = base_row + WE
        pltpu.sync_copy(z_hbm, zbuf)
        bufs = ((x0, id0, ix0, s_x0, s_id0), (x1, id1, ix1, s_x1, s_id1))

        def window_body(kk, carry):
            win = wlo + w + NWORK * kk

            @pl.when(win < wlo + span)
            def _():
                e0 = win * WE
                pltpu.sync_copy(p2_hbm.at[win], pvec)
                p_start = pvec[0, :][0]
                p_end = pvec[1, :][0]
                a0 = (p_start // C) * C
                nchunks = jnp.where(p_end > p_start,
                                    (p_end - a0 + C - 1) // C, 0)
                for r in range(ROWS // ZR):
                    pltpu.sync_copy(zbuf, acc.at[pl.ds(base_row + r * ZR, ZR)])

                def chunk_pc(j):
                    p = a0 + j * C
                    return p, pl.multiple_of(jnp.minimum(p, NP - C), C)

                def start_load(j, b):
                    xb, idb, _, sx, sid_sem = bufs[b]
                    _, pc = chunk_pc(j)
                    pltpu.async_copy(ids_hbm.at[pl.ds(pc, C)], idb, sid_sem)
                    pltpu.async_copy(xq_hbm.at[pl.ds(pc, C)], xb, sx)

                def process(j, b):
                    xb, idb, ixb, sx, sid_sem = bufs[b]
                    p, pc = chunk_pc(j)
                    pltpu.make_async_copy(ids_hbm.at[pl.ds(0, C)], idb,
                                          sid_sem).wait()
                    lo = jnp.maximum(p, p_start)
                    for g in range(C // 16):
                        idv = idb[pl.ds(g * 16, 16)]
                        gi = pc + g * 16 + lax.iota(jnp.int32, 16)
                        valid = (gi >= lo) & (gi < p_end)
                        loc = jnp.where(valid, idv - e0 + base_row, dump)
                        ixb[pl.ds(g * 16, 16)] = loc
                    pltpu.make_async_copy(xq_hbm.at[pl.ds(0, C)], xb, sx).wait()
                    pltpu.sync_copy(xb, acc.at[ixb], add=True)

                @pl.when(nchunks > 0)
                def _():
                    start_load(0, 0)

                    def pair_body(t, carry2):
                        c0 = 2 * t
                        c1 = c0 + 1
                        c2 = c0 + 2

                        @pl.when(c1 < nchunks)
                        def _():
                            start_load(c1, 1)

                        process(c0, 0)

                        @pl.when(c2 < nchunks)
                        def _():
                            start_load(c2, 0)

                        @pl.when(c1 < nchunks)
                        def _():
                            process(c1, 1)

                        return carry2

                    lax.fori_loop(0, (nchunks + 1) // 2, pair_body, 0)

                pltpu.sync_copy(acc.at[pl.ds(base_row, WE)],
                                agg_hbm.at[pl.ds(e0 - wlo * WE, WE)])

            return carry

        lax.fori_loop(0, wpw, window_body, 0)

    return k(x_q, ids, p2, zrows)


def _tc_combine_slice(agg, x_edge, w_q, row0, nrows, prev=None):
    R = 1024 if nrows % 1024 == 0 and row0 % 1024 == 0 else 512
    off = row0 // R

    def body(agg_ref, xe_ref, wq_ref, *rest):
        out_ref = rest[-1]
        out_ref[...] = ALPHA_C * xe_ref[...] + (1.0 - ALPHA_C) * jnp.dot(
            agg_ref[...], wq_ref[...], preferred_element_type=jnp.float32)

    in_specs = [
        pl.BlockSpec((R, D), lambda i: (i, 0)),
        pl.BlockSpec((R, D), lambda i, _off=off: (i + _off, 0)),
        pl.BlockSpec((D, D), lambda i: (0, 0)),
    ]
    args = [agg, x_edge, w_q]
    aliases = {}
    if prev is not None:
        in_specs.append(pl.BlockSpec(memory_space=pl.ANY))
        args.append(prev)
        aliases = {3: 0}
    return pl.pallas_call(
        body,
        grid=(nrows // R,),
        in_specs=in_specs,
        out_specs=pl.BlockSpec((R, D), lambda i, _off=off: (i + _off, 0)),
        out_shape=jax.ShapeDtypeStruct((NE, D), jnp.float32),
        input_output_aliases=aliases,
    )(*args)


def kernel(x_q, x_edge, edge_ids, w_q):
    ids = edge_ids.astype(jnp.int32)
    bounds = jnp.arange(NWIN + 1, dtype=jnp.int32) * WE
    stride = 128
    s = ids[::stride]
    k = jnp.sum(s[None, :] < bounds[:, None], axis=1).astype(jnp.int32)
    w0 = jnp.clip((k - 1) * stride, 0, NP - stride)
    wnd = ids[w0[:, None] + jnp.arange(stride, dtype=jnp.int32)[None, :]]
    p = (w0 + jnp.sum(wnd < bounds[:, None], axis=1)).astype(jnp.int32)
    p2 = jnp.stack([
        jnp.broadcast_to(p[:-1, None], (NWIN, 16)),
        jnp.broadcast_to(p[1:, None], (NWIN, 16)),
    ], axis=1)
    z = jnp.zeros((ZR, D), jnp.float32)
    splits = [0, 156, 312, 468, NWIN]
    aggs = [
        _sc_segment_sum(x_q, ids, p2, z, splits[j], splits[j + 1] - splits[j])
        for j in range(len(splits) - 1)
    ]
    out = None
    for j, agg in enumerate(aggs):
        out = _tc_combine_slice(agg, x_edge, w_q, splits[j] * WE,
                                (splits[j + 1] - splits[j]) * WE, out)
    return out


# speedup vs baseline: 4.8367x; 1.0510x over previous
import functools

import jax
import jax.numpy as jnp
from jax import lax
from jax.experimental import pallas as pl
from jax.experimental.pallas import tpu as pltpu
from jax.experimental.pallas import tpu_sc as plsc

NP = 640000
NE = 320000
D = 128
ALPHA_C = 0.5

WE = 512
NWIN = NE // WE
NC, NS = 2, 16
NWORK = NC * NS
WPW = (NWIN + NWORK - 1) // NWORK
C = 128
ROWS = WE + 8
ZR = ROWS // 8


def _sc_segment_sum(x_q, ids, p2, zrows, wlo, span):
    mesh = plsc.VectorSubcoreMesh(core_axis_name="c", subcore_axis_name="s")
    wpw = (span + NWORK - 1) // NWORK

    @functools.partial(
        pl.kernel,
        out_type=jax.ShapeDtypeStruct((span * WE, D), jnp.float32),
        mesh=mesh,
        scratch_types=(
            [pltpu.VMEM((C, D), jnp.float32) for _ in range(3)]
            + [pltpu.VMEM((C,), jnp.int32) for _ in range(3)]
            + [pltpu.VMEM((C,), jnp.int32) for _ in range(3)]
            + [
                pltpu.VMEM((2, 16), jnp.int32),
                pltpu.VMEM((ZR, D), jnp.float32),
                pltpu.VMEM_SHARED((NS * ROWS, D), jnp.float32),
            ]
            + [pltpu.SemaphoreType.DMA for _ in range(9)]
        ),
    )
    def k(xq_hbm, ids_hbm, p2_hbm, z_hbm, agg_hbm,
          x0, x1, x2, id0, id1, id2, ix0, ix1, ix2, pvec, zbuf, acc,
          s_x0, s_x1, s_x2, s_id0, s_id1, s_id2, s_sc0, s_sc1, s_sc2):
        cid = lax.axis_index("c")
        sid = lax.axis_index("s")
        w = cid * NS + sid
        base_row = sid * ROWS
        dump = base_row + WE
        pltpu.sync_copy(z_hbm, zbuf)
        bufs = ((x0, id0, ix0, s_x0, s_id0, s_sc0),
                (x1, id1, ix1, s_x1, s_id1, s_sc1),
                (x2, id2, ix2, s_x2, s_id2, s_sc2))

        def window_body(kk, carry):
            win = wlo + w + NWORK * kk

            @pl.when(win < wlo + span)
            def _():
                e0 = win * WE
                pltpu.sync_copy(p2_hbm.at[win], pvec)
                p_start = pvec[0, :][0]
                p_end = pvec[1, :][0]
                a0 = (p_start // C) * C
                nchunks = jnp.where(p_end > p_start,
                                    (p_end - a0 + C - 1) // C, 0)
                for r in range(ROWS // ZR):
                    pltpu.sync_copy(zbuf, acc.at[pl.ds(base_row + r * ZR, ZR)])

                def chunk_pc(j):
                    p = a0 + j * C
                    return p, pl.multiple_of(jnp.minimum(p, NP - C), C)

                def start_load(j, b):
                    xb, idb, _, sx, sid_sem, _sc = bufs[b]
                    _, pc = chunk_pc(j)
                    pltpu.async_copy(ids_hbm.at[pl.ds(pc, C)], idb, sid_sem)
                    pltpu.async_copy(xq_hbm.at[pl.ds(pc, C)], xb, sx)

                def prep(j, b):
                    xb, idb, ixb, sx, sid_sem, s_sc = bufs[b]
                    p, pc = chunk_pc(j)
                    pltpu.make_async_copy(ids_hbm.at[pl.ds(0, C)], idb,
                                          sid_sem).wait()
                    lo = jnp.maximum(p, p_start)
                    for g in range(C // 16):
                        idv = idb[pl.ds(g * 16, 16)]
                        gi = pc + g * 16 + lax.iota(jnp.int32, 16)
                        valid = (gi >= lo) & (gi < p_end)
                        loc = jnp.where(valid, idv - e0 + base_row, dump)
                        ixb[pl.ds(g * 16, 16)] = loc
                    pltpu.make_async_copy(xq_hbm.at[pl.ds(0, C)], xb, sx).wait()

                def start_scat(j, b):
                    xb, _, ixb, _, _, s_sc = bufs[b]
                    pltpu.async_copy(xb, acc.at[ixb], s_sc, add=True)

                def wait_scat(b):
                    xb, _, ixb, _, _, s_sc = bufs[b]
                    pltpu.make_async_copy(xb, acc.at[ixb], s_sc).wait()

                @pl.when(nchunks > 0)
                def _():
                    start_load(0, 0)

                    @pl.when(nchunks > 1)
                    def _():
                        start_load(1, 1)

                    def tri_body(t, carry2):
                        for u in range(3):
                            c = 3 * t + u

                            @pl.when(c < nchunks)
                            def _(c=c, u=u):
                                prep(c, u)

                                @pl.when(c >= 1)
                                def _():
                                    wait_scat((u + 2) % 3)

                                start_scat(c, u)

                                @pl.when(c + 2 < nchunks)
                                def _():
                                    start_load(c + 2, (u + 2) % 3)

                        return carry2

                    lax.fori_loop(0, (nchunks + 2) // 3, tri_body, 0)
                    for u in range(3):
                        @pl.when((nchunks - 1) % 3 == u)
                        def _(u=u):
                            wait_scat(u)

                pltpu.sync_copy(acc.at[pl.ds(base_row, WE)],
                                agg_hbm.at[pl.ds(e0 - wlo * WE, WE)])

            return carry

        lax.fori_loop(0, wpw, window_body, 0)

    return k(x_q, ids, p2, zrows)


def _tc_combine_slice(agg, x_edge, w_q, row0, nrows, prev=None):
    R = 1024 if nrows % 1024 == 0 and row0 % 1024 == 0 else 512
    off = row0 // R

    def body(agg_ref, xe_ref, wq_ref, *rest):
        out_ref = rest[-1]
        out_ref[...] = ALPHA_C * xe_ref[...] + (1.0 - ALPHA_C) * jnp.dot(
            agg_ref[...], wq_ref[...], preferred_element_type=jnp.float32)

    in_specs = [
        pl.BlockSpec((R, D), lambda i: (i, 0)),
        pl.BlockSpec((R, D), lambda i, _off=off: (i + _off, 0)),
        pl.BlockSpec((D, D), lambda i: (0, 0)),
    ]
    args = [agg, x_edge, w_q]
    aliases = {}
    if prev is not None:
        in_specs.append(pl.BlockSpec(memory_space=pl.ANY))
        args.append(prev)
        aliases = {3: 0}
    return pl.pallas_call(
        body,
        grid=(nrows // R,),
        in_specs=in_specs,
        out_specs=pl.BlockSpec((R, D), lambda i, _off=off: (i + _off, 0)),
        out_shape=jax.ShapeDtypeStruct((NE, D), jnp.float32),
        input_output_aliases=aliases,
    )(*args)


def kernel(x_q, x_edge, edge_ids, w_q):
    ids = edge_ids.astype(jnp.int32)
    bounds = jnp.arange(NWIN + 1, dtype=jnp.int32) * WE
    stride = 128
    s = ids[::stride]
    k = jnp.sum(s[None, :] < bounds[:, None], axis=1).astype(jnp.int32)
    w0 = jnp.clip((k - 1) * stride, 0, NP - stride)
    wnd = ids[w0[:, None] + jnp.arange(stride, dtype=jnp.int32)[None, :]]
    p = (w0 + jnp.sum(wnd < bounds[:, None], axis=1)).astype(jnp.int32)
    p2 = jnp.stack([
        jnp.broadcast_to(p[:-1, None], (NWIN, 16)),
        jnp.broadcast_to(p[1:, None], (NWIN, 16)),
    ], axis=1)
    z = jnp.zeros((ZR, D), jnp.float32)
    splits = [0, 156, 312, 468, NWIN]
    aggs = [
        _sc_segment_sum(x_q, ids, p2, z, splits[j], splits[j + 1] - splits[j])
        for j in range(len(splits) - 1)
    ]
    out = None
    for j, agg in enumerate(aggs):
        out = _tc_combine_slice(agg, x_edge, w_q, splits[j] * WE,
                                (splits[j + 1] - splits[j]) * WE, out)
    return out


# speedup vs baseline: 4.8526x; 1.0033x over previous
import functools

import jax
import jax.numpy as jnp
from jax import lax
from jax.experimental import pallas as pl
from jax.experimental.pallas import tpu as pltpu
from jax.experimental.pallas import tpu_sc as plsc

NP = 640000
NE = 320000
D = 128
ALPHA_C = 0.5

WE = 512
NWIN = NE // WE
NC, NS = 2, 16
NWORK = NC * NS
WPW = (NWIN + NWORK - 1) // NWORK
C = 128
ROWS = WE + 8
ZR = ROWS // 8


def _sc_segment_sum(x_q, ids, p2, zrows, wlo, span):
    mesh = plsc.VectorSubcoreMesh(core_axis_name="c", subcore_axis_name="s")
    wpw = (span + NWORK - 1) // NWORK

    @functools.partial(
        pl.kernel,
        out_type=jax.ShapeDtypeStruct((span * WE, D), jnp.float32),
        mesh=mesh,
        scratch_types=(
            [pltpu.VMEM((C, D), jnp.float32) for _ in range(3)]
            + [pltpu.VMEM((C,), jnp.int32) for _ in range(3)]
            + [pltpu.VMEM((C,), jnp.int32) for _ in range(3)]
            + [
                pltpu.VMEM((2, 16), jnp.int32),
                pltpu.VMEM((ZR, D), jnp.float32),
                pltpu.VMEM_SHARED((NS * ROWS, D), jnp.float32),
            ]
            + [pltpu.SemaphoreType.DMA for _ in range(10)]
        ),
    )
    def k(xq_hbm, ids_hbm, p2_hbm, z_hbm, agg_hbm,
          x0, x1, x2, id0, id1, id2, ix0, ix1, ix2, pvec, zbuf, acc,
          s_x0, s_x1, s_x2, s_id0, s_id1, s_id2, s_sc0, s_sc1, s_sc2, s_z):
        cid = lax.axis_index("c")
        sid = lax.axis_index("s")
        w = cid * NS + sid
        base_row = sid * ROWS
        dump = base_row + WE
        pltpu.sync_copy(z_hbm, zbuf)
        bufs = ((x0, id0, ix0, s_x0, s_id0, s_sc0),
                (x1, id1, ix1, s_x1, s_id1, s_sc1),
                (x2, id2, ix2, s_x2, s_id2, s_sc2))

        def fire_zeros():
            for r in range(ROWS // ZR):
                pltpu.async_copy(zbuf, acc.at[pl.ds(base_row + r * ZR, ZR)],
                                 s_z)

        def drain_zeros():
            for r in range(ROWS // ZR):
                pltpu.make_async_copy(zbuf, acc.at[pl.ds(base_row, ZR)],
                                      s_z).wait()

        fire_zeros()

        def window_body(kk, carry):
            win = wlo + w + NWORK * kk

            @pl.when(win < wlo + span)
            def _():
                e0 = win * WE
                lw = win - wlo
                pltpu.sync_copy(p2_hbm.at[lw], pvec)
                p_start = pvec[0, :][0]
                p_end = pvec[1, :][0]
                a0 = (p_start // C) * C
                nchunks = jnp.where(p_end > p_start,
                                    (p_end - a0 + C - 1) // C, 0)

                def chunk_pc(j):
                    p = a0 + j * C
                    return p, pl.multiple_of(jnp.minimum(p, NP - C), C)

                def start_load(j, b):
                    xb, idb, _, sx, sid_sem, _sc = bufs[b]
                    _, pc = chunk_pc(j)
                    pltpu.async_copy(ids_hbm.at[pl.ds(pc, C)], idb, sid_sem)
                    pltpu.async_copy(xq_hbm.at[pl.ds(pc, C)], xb, sx)

                def prep(j, b):
                    xb, idb, ixb, sx, sid_sem, s_sc = bufs[b]
                    p, pc = chunk_pc(j)
                    pltpu.make_async_copy(ids_hbm.at[pl.ds(0, C)], idb,
                                          sid_sem).wait()
                    lo = jnp.maximum(p, p_start)
                    for g in range(C // 16):
                        idv = idb[pl.ds(g * 16, 16)]
                        gi = pc + g * 16 + lax.iota(jnp.int32, 16)
                        valid = (gi >= lo) & (gi < p_end)
                        loc = jnp.where(valid, idv - e0 + base_row, dump)
                        ixb[pl.ds(g * 16, 16)] = loc
                    pltpu.make_async_copy(xq_hbm.at[pl.ds(0, C)], xb, sx).wait()

                def start_scat(j, b):
                    xb, _, ixb, _, _, s_sc = bufs[b]
                    pltpu.async_copy(xb, acc.at[ixb], s_sc, add=True)

                def wait_scat(b):
                    xb, _, ixb, _, _, s_sc = bufs[b]
                    pltpu.make_async_copy(xb, acc.at[ixb], s_sc).wait()

                @pl.when(nchunks > 0)
                def _():
                    start_load(0, 0)

                    @pl.when(nchunks > 1)
                    def _():
                        start_load(1, 1)

                drain_zeros()

                @pl.when(nchunks > 0)
                def _():
                    def tri_body(t, carry2):
                        for u in range(3):
                            c = 3 * t + u

                            @pl.when(c < nchunks)
                            def _(c=c, u=u):
                                prep(c, u)

                                @pl.when(c >= 1)
                                def _():
                                    wait_scat((u + 2) % 3)

                                start_scat(c, u)

                                @pl.when(c + 2 < nchunks)
                                def _():
                                    start_load(c + 2, (u + 2) % 3)

                        return carry2

                    lax.fori_loop(0, (nchunks + 2) // 3, tri_body, 0)
                    for u in range(3):
                        @pl.when((nchunks - 1) % 3 == u)
                        def _(u=u):
                            wait_scat(u)

                pltpu.sync_copy(acc.at[pl.ds(base_row, WE)],
                                agg_hbm.at[pl.ds(lw * WE, WE)])
                fire_zeros()

            return carry

        lax.fori_loop(0, wpw, window_body, 0)
        drain_zeros()

    return k(x_q, ids, p2, zrows)


def _tc_combine_slice(agg, x_edge, w_q, row0, nrows, prev=None):
    R = 1024 if nrows % 1024 == 0 and row0 % 1024 == 0 else 512
    off = row0 // R

    def body(agg_ref, xe_ref, wq_ref, *rest):
        out_ref = rest[-1]
        out_ref[...] = ALPHA_C * xe_ref[...] + (1.0 - ALPHA_C) * jnp.dot(
            agg_ref[...], wq_ref[...], preferred_element_type=jnp.float32)

    in_specs = [
        pl.BlockSpec((R, D), lambda i: (i, 0)),
        pl.BlockSpec((R, D), lambda i, _off=off: (i + _off, 0)),
        pl.BlockSpec((D, D), lambda i: (0, 0)),
    ]
    args = [agg, x_edge, w_q]
    aliases = {}
    if prev is not None:
        in_specs.append(pl.BlockSpec(memory_space=pl.ANY))
        args.append(prev)
        aliases = {3: 0}
    return pl.pallas_call(
        body,
        grid=(nrows // R,),
        in_specs=in_specs,
        out_specs=pl.BlockSpec((R, D), lambda i, _off=off: (i + _off, 0)),
        out_shape=jax.ShapeDtypeStruct((NE, D), jnp.float32),
        input_output_aliases=aliases,
    )(*args)


def kernel(x_q, x_edge, edge_ids, w_q):
    ids = edge_ids.astype(jnp.int32)
    stride = 128
    s = ids[::stride]
    z = jnp.zeros((ZR, D), jnp.float32)

    def slice_p2(wlo, span):
        bounds = (wlo + jnp.arange(span + 1, dtype=jnp.int32)) * WE
        k = jnp.sum(s[None, :] < bounds[:, None], axis=1).astype(jnp.int32)
        w0 = jnp.clip((k - 1) * stride, 0, NP - stride)
        wnd = ids[w0[:, None] + jnp.arange(stride, dtype=jnp.int32)[None, :]]
        p = (w0 + jnp.sum(wnd < bounds[:, None], axis=1)).astype(jnp.int32)
        return jnp.stack([
            jnp.broadcast_to(p[:-1, None], (span, 16)),
            jnp.broadcast_to(p[1:, None], (span, 16)),
        ], axis=1)

    splits = [0, 168, 336, 504, NWIN]
    aggs = [
        _sc_segment_sum(x_q, ids, slice_p2(splits[j],
                                           splits[j + 1] - splits[j]),
                        z, splits[j], splits[j + 1] - splits[j])
        for j in range(len(splits) - 1)
    ]
    out = None
    for j, agg in enumerate(aggs):
        out = _tc_combine_slice(agg, x_edge, w_q, splits[j] * WE,
                                (splits[j + 1] - splits[j]) * WE, out)
    return out
